# Initial kernel scaffold; baseline (speedup 1.0000x reference)
import jax
import jax.numpy as jnp
from jax.experimental import pallas as pl


def kernel(node_features, batch):
    raise NotImplementedError("write your pallas kernel here")


# speedup vs baseline: 2.3290x; 2.3290x over previous
import functools

import jax
import jax.numpy as jnp
from jax import lax
from jax.experimental import pallas as pl
from jax.experimental.pallas import tpu as pltpu
from jax.experimental.pallas import tpu_sc as plsc

N_ROWS = 100000
D = 128
N_SEG = 256
LANES = 16
CHUNK = 160
N_CHUNKS = N_ROWS // CHUNK
N_CORES = 2
N_SUBCORES = 16
N_WORKERS = N_CORES * N_SUBCORES
MAX_CHUNKS_PER_W = -(-N_CHUNKS // N_WORKERS)
IDX_ROWS = CHUNK // LANES


def _sc_partial_sums(node_features, batch2d):
    mesh = plsc.VectorSubcoreMesh(core_axis_name="c", subcore_axis_name="s")

    @functools.partial(
        pl.kernel,
        out_type=jax.ShapeDtypeStruct((N_WORKERS, N_SEG, D), jnp.float32),
        mesh=mesh,
        scratch_types=[
            pltpu.VMEM((CHUNK, D), jnp.float32),
            pltpu.VMEM((CHUNK,), jnp.int32),
            pltpu.VMEM((N_SEG, D), jnp.float32),
            pltpu.SemaphoreType.DMA,
        ],
    )
    def body(x_hbm, b_hbm, out_hbm, xbuf, ibuf, acc, sem):
        cid = lax.axis_index("c")
        sid = lax.axis_index("s")
        wid = sid * N_CORES + cid

        zero = jnp.zeros((LANES,), jnp.float32)

        def zero_row(r, carry):
            for k in range(D // LANES):
                acc[r, pl.ds(k * LANES, LANES)] = zero
            return carry

        lax.fori_loop(0, N_SEG, zero_row, 0)

        def do_chunk(j, carry):
            chunk_id = wid + j * N_WORKERS

            @pl.when(chunk_id < N_CHUNKS)
            def _():
                base = chunk_id * CHUNK
                pltpu.sync_copy(x_hbm.at[pl.ds(base, CHUNK)], xbuf)
                pltpu.sync_copy(b_hbm.at[pl.ds(base, CHUNK)], ibuf)

                def do_group(g, c2):
                    segs = ibuf[pl.ds(g * LANES, LANES)]
                    for l in range(LANES):
                        s = segs[l]
                        for k in range(D // LANES):
                            sl = pl.ds(k * LANES, LANES)
                            plsc.addupdate(
                                acc.at[s, sl], xbuf[g * LANES + l, sl]
                            )
                    return c2

                lax.fori_loop(0, CHUNK // LANES, do_group, 0)

            return carry

        lax.fori_loop(0, MAX_CHUNKS_PER_W, do_chunk, 0)

        pltpu.sync_copy(acc, out_hbm.at[wid])

    return body(node_features, batch2d)


def _combine_body(p_ref, o_ref):
    o_ref[...] = jnp.sum(p_ref[...], axis=0)


def _combine(partials):
    return pl.pallas_call(
        _combine_body,
        out_shape=jax.ShapeDtypeStruct((N_SEG, D), jnp.float32),
    )(partials)


def kernel(node_features, batch):
    partials = _sc_partial_sums(node_features, batch.astype(jnp.int32))
    return _combine(partials)


# speedup vs baseline: 3.0246x; 1.2987x over previous
import functools

import jax
import jax.numpy as jnp
from jax import lax
from jax.experimental import pallas as pl
from jax.experimental.pallas import tpu as pltpu
from jax.experimental.pallas import tpu_sc as plsc

N_ROWS = 100000
D = 128
N_SEG = 256
LANES = 16
CHUNK = 160
N_CHUNKS = N_ROWS // CHUNK
N_CORES = 2
N_SUBCORES = 16
N_WORKERS = N_CORES * N_SUBCORES
MAX_CHUNKS_PER_W = -(-N_CHUNKS // N_WORKERS)
IDX_ROWS = CHUNK // LANES


def _sc_partial_sums(node_features, batch2d):
    mesh = plsc.VectorSubcoreMesh(core_axis_name="c", subcore_axis_name="s")

    @functools.partial(
        pl.kernel,
        out_type=jax.ShapeDtypeStruct((N_WORKERS, N_SEG, D), jnp.float32),
        mesh=mesh,
        scratch_types=[
            pltpu.VMEM((CHUNK, D), jnp.float32),
            pltpu.VMEM((CHUNK, D), jnp.float32),
            pltpu.VMEM((CHUNK,), jnp.int32),
            pltpu.VMEM((CHUNK,), jnp.int32),
            pltpu.VMEM((N_SEG, D), jnp.float32),
            pltpu.SemaphoreType.DMA,
            pltpu.SemaphoreType.DMA,
            pltpu.SemaphoreType.DMA,
            pltpu.SemaphoreType.DMA,
        ],
    )
    def body(x_hbm, b_hbm, out_hbm, xb0, xb1, ib0, ib1, acc, sx0, si0, sx1, si1):
        cid = lax.axis_index("c")
        sid = lax.axis_index("s")
        wid = sid * N_CORES + cid
        xbuf = (xb0, xb1)
        ibuf = (ib0, ib1)
        semx = (sx0, sx1)
        semi = (si0, si1)

        zero = jnp.zeros((LANES,), jnp.float32)

        def zero_row(r, carry):
            for k in range(D // LANES):
                acc[r, pl.ds(k * LANES, LANES)] = zero
            return carry

        lax.fori_loop(0, N_SEG, zero_row, 0)

        def copies(j, slot):
            chunk_id = wid + j * N_WORKERS
            base = chunk_id * CHUNK
            xc = pltpu.make_async_copy(
                x_hbm.at[pl.ds(base, CHUNK)], xbuf[slot], semx[slot]
            )
            ic = pltpu.make_async_copy(
                b_hbm.at[pl.ds(base, CHUNK)], ibuf[slot], semi[slot]
            )
            return chunk_id, xc, ic

        def start(j, slot):
            chunk_id, xc, ic = copies(j, slot)

            @pl.when(chunk_id < N_CHUNKS)
            def _():
                xc.start()
                ic.start()

        def wait_and_compute(j, slot):
            chunk_id, xc, ic = copies(j, slot)

            @pl.when(chunk_id < N_CHUNKS)
            def _():
                xc.wait()
                ic.wait()

                def do_group(g, c2):
                    segs = ibuf[slot][pl.ds(g * LANES, LANES)]
                    for l in range(LANES):
                        s = segs[l]
                        for k in range(D // LANES):
                            sl = pl.ds(k * LANES, LANES)
                            plsc.addupdate(
                                acc.at[s, sl], xbuf[slot][g * LANES + l, sl]
                            )
                    return c2

                lax.fori_loop(0, CHUNK // LANES, do_group, 0)

        start(0, 0)

        def pair(j2, carry):
            j0 = 2 * j2
            start(j0 + 1, 1)
            wait_and_compute(j0, 0)
            start(j0 + 2, 0)
            wait_and_compute(j0 + 1, 1)
            return carry

        lax.fori_loop(0, MAX_CHUNKS_PER_W // 2, pair, 0)

        pltpu.sync_copy(acc, out_hbm.at[wid])

    return body(node_features, batch2d)


def _combine_body(p_ref, o_ref):
    o_ref[...] = jnp.sum(p_ref[...], axis=0)


def _combine(partials):
    return pl.pallas_call(
        _combine_body,
        out_shape=jax.ShapeDtypeStruct((N_SEG, D), jnp.float32),
    )(partials)


def kernel(node_features, batch):
    partials = _sc_partial_sums(node_features, batch.astype(jnp.int32))
    return _combine(partials)


# speedup vs baseline: 5.0971x; 1.6852x over previous
import functools

import jax
import jax.numpy as jnp
from jax import lax
from jax.experimental import pallas as pl
from jax.experimental.pallas import tpu as pltpu
from jax.experimental.pallas import tpu_sc as plsc

N_ROWS = 100000
D = 128
N_SEG = 256
LANES = 16
CHUNK = 160
N_CHUNKS = N_ROWS // CHUNK
N_CORES = 2
N_SUBCORES = 16
N_WORKERS = N_CORES * N_SUBCORES
MAX_CHUNKS_PER_W = -(-N_CHUNKS // N_WORKERS)
IDX_ROWS = CHUNK // LANES


def _sc_partial_sums(node_features, batch2d):
    mesh = plsc.VectorSubcoreMesh(core_axis_name="c", subcore_axis_name="s")

    @functools.partial(
        pl.kernel,
        out_type=jax.ShapeDtypeStruct((N_WORKERS, N_SEG, D), jnp.float32),
        mesh=mesh,
        scratch_types=[
            pltpu.VMEM((CHUNK, D), jnp.float32),
            pltpu.VMEM((CHUNK, D), jnp.float32),
            pltpu.VMEM((CHUNK,), jnp.int32),
            pltpu.VMEM((CHUNK,), jnp.int32),
            pltpu.VMEM((N_SEG, D), jnp.float32),
            pltpu.SemaphoreType.DMA,
            pltpu.SemaphoreType.DMA,
            pltpu.SemaphoreType.DMA,
            pltpu.SemaphoreType.DMA,
        ],
    )
    def body(x_hbm, b_hbm, out_hbm, xb0, xb1, ib0, ib1, acc, sx0, si0, sx1, si1):
        cid = lax.axis_index("c")
        sid = lax.axis_index("s")
        wid = sid * N_CORES + cid
        xbuf = (xb0, xb1)
        ibuf = (ib0, ib1)
        semx = (sx0, sx1)
        semi = (si0, si1)

        zero = jnp.zeros((LANES,), jnp.float32)

        def zero_row(r, carry):
            for k in range(D // LANES):
                acc[r, pl.ds(k * LANES, LANES)] = zero
            return carry

        lax.fori_loop(0, N_SEG, zero_row, 0)

        def copies(j, slot):
            chunk_id = wid + j * N_WORKERS
            base = chunk_id * CHUNK
            xc = pltpu.make_async_copy(
                x_hbm.at[pl.ds(base, CHUNK)], xbuf[slot], semx[slot]
            )
            ic = pltpu.make_async_copy(
                b_hbm.at[pl.ds(base, CHUNK)], ibuf[slot], semi[slot]
            )
            return chunk_id, xc, ic

        def start(j, slot):
            chunk_id, xc, ic = copies(j, slot)

            @pl.when(chunk_id < N_CHUNKS)
            def _():
                xc.start()
                ic.start()

        def wait_and_compute(j, slot):
            chunk_id, xc, ic = copies(j, slot)

            @pl.when(chunk_id < N_CHUNKS)
            def _():
                xc.wait()
                ic.wait()

                def do_group(g, c2):
                    segs = ibuf[slot][pl.ds(g * LANES, LANES)]
                    s0 = segs[0]
                    row0 = g * LANES

                    def fast():
                        for k in range(D // LANES):
                            sl = pl.ds(k * LANES, LANES)
                            v = xbuf[slot][row0, sl]
                            for l in range(1, LANES):
                                v = v + xbuf[slot][row0 + l, sl]
                            plsc.addupdate(acc.at[s0, sl], v)

                    def slow():
                        for l in range(LANES):
                            s = segs[l]
                            for k in range(D // LANES):
                                sl = pl.ds(k * LANES, LANES)
                                plsc.addupdate(
                                    acc.at[s, sl], xbuf[slot][row0 + l, sl]
                                )

                    lax.cond(s0 == segs[LANES - 1], fast, slow)
                    return c2

                lax.fori_loop(0, CHUNK // LANES, do_group, 0)

        start(0, 0)

        def pair(j2, carry):
            j0 = 2 * j2
            start(j0 + 1, 1)
            wait_and_compute(j0, 0)
            start(j0 + 2, 0)
            wait_and_compute(j0 + 1, 1)
            return carry

        lax.fori_loop(0, MAX_CHUNKS_PER_W // 2, pair, 0)

        pltpu.sync_copy(acc, out_hbm.at[wid])

    return body(node_features, batch2d)


def _combine_body(p_ref, o_ref):
    o_ref[...] = jnp.sum(p_ref[...], axis=0)


def _combine(partials):
    return pl.pallas_call(
        _combine_body,
        out_shape=jax.ShapeDtypeStruct((N_SEG, D), jnp.float32),
    )(partials)


def kernel(node_features, batch):
    partials = _sc_partial_sums(node_features, batch.astype(jnp.int32))
    return _combine(partials)


# speedup vs baseline: 5.6866x; 1.1157x over previous
import functools

import jax
import jax.numpy as jnp
from jax import lax
from jax.experimental import pallas as pl
from jax.experimental.pallas import tpu as pltpu
from jax.experimental.pallas import tpu_sc as plsc

N_ROWS = 100000
D = 128
N_SEG = 256
LANES = 16
CHUNK = 160
N_CHUNKS = N_ROWS // CHUNK
N_CORES = 2
N_SUBCORES = 16
N_WORKERS = N_CORES * N_SUBCORES
MAX_CHUNKS_PER_W = -(-N_CHUNKS // N_WORKERS)
IDX_ROWS = CHUNK // LANES


def _sc_partial_sums(node_features, batch2d):
    mesh = plsc.VectorSubcoreMesh(core_axis_name="c", subcore_axis_name="s")

    @functools.partial(
        pl.kernel,
        out_type=jax.ShapeDtypeStruct((N_WORKERS, N_SEG, D), jnp.float32),
        mesh=mesh,
        scratch_types=[
            pltpu.VMEM((CHUNK, D), jnp.float32),
            pltpu.VMEM((CHUNK, D), jnp.float32),
            pltpu.VMEM((CHUNK,), jnp.int32),
            pltpu.VMEM((CHUNK,), jnp.int32),
            pltpu.VMEM((N_SEG, D), jnp.float32),
            pltpu.SemaphoreType.DMA,
            pltpu.SemaphoreType.DMA,
            pltpu.SemaphoreType.DMA,
            pltpu.SemaphoreType.DMA,
        ],
    )
    def body(x_hbm, b_hbm, out_hbm, xb0, xb1, ib0, ib1, acc, sx0, si0, sx1, si1):
        cid = lax.axis_index("c")
        sid = lax.axis_index("s")
        wid = sid * N_CORES + cid
        xbuf = (xb0, xb1)
        ibuf = (ib0, ib1)
        semx = (sx0, sx1)
        semi = (si0, si1)

        zero = jnp.zeros((LANES,), jnp.float32)

        def zero_row(r, carry):
            for k in range(D // LANES):
                acc[r, pl.ds(k * LANES, LANES)] = zero
            return carry

        lax.fori_loop(0, N_SEG, zero_row, 0)

        def copies(j, slot):
            chunk_id = wid + j * N_WORKERS
            base = chunk_id * CHUNK
            xc = pltpu.make_async_copy(
                x_hbm.at[pl.ds(base, CHUNK)], xbuf[slot], semx[slot]
            )
            ic = pltpu.make_async_copy(
                b_hbm.at[pl.ds(base, CHUNK)], ibuf[slot], semi[slot]
            )
            return chunk_id, xc, ic

        def start(j, slot):
            chunk_id, xc, ic = copies(j, slot)

            @pl.when(chunk_id < N_CHUNKS)
            def _():
                xc.start()
                ic.start()

        def wait_and_compute(j, slot):
            chunk_id, xc, ic = copies(j, slot)

            @pl.when(chunk_id < N_CHUNKS)
            def _():
                xc.wait()
                ic.wait()

                def do_group(g, c2):
                    segs = ibuf[slot][pl.ds(g * LANES, LANES)]
                    s0 = segs[0]
                    row0 = g * LANES

                    def fast():
                        for k in range(D // LANES):
                            sl = pl.ds(k * LANES, LANES)
                            vals = [
                                xbuf[slot][row0 + l, sl] for l in range(LANES)
                            ]
                            while len(vals) > 1:
                                vals = [
                                    vals[i] + vals[i + 1]
                                    for i in range(0, len(vals), 2)
                                ]
                            plsc.addupdate(acc.at[s0, sl], vals[0])

                    def slow():
                        for l in range(LANES):
                            s = segs[l]
                            for k in range(D // LANES):
                                sl = pl.ds(k * LANES, LANES)
                                plsc.addupdate(
                                    acc.at[s, sl], xbuf[slot][row0 + l, sl]
                                )

                    lax.cond(s0 == segs[LANES - 1], fast, slow)
                    return c2

                lax.fori_loop(0, CHUNK // LANES, do_group, 0)

        start(0, 0)

        def pair(j2, carry):
            j0 = 2 * j2
            start(j0 + 1, 1)
            wait_and_compute(j0, 0)
            start(j0 + 2, 0)
            wait_and_compute(j0 + 1, 1)
            return carry

        lax.fori_loop(0, MAX_CHUNKS_PER_W // 2, pair, 0)

        pltpu.sync_copy(acc, out_hbm.at[wid])

    return body(node_features, batch2d)


def _combine_body(p_ref, o_ref):
    o_ref[...] = jnp.sum(p_ref[...], axis=0)


def _combine(partials):
    return pl.pallas_call(
        _combine_body,
        out_shape=jax.ShapeDtypeStruct((N_SEG, D), jnp.float32),
    )(partials)


def kernel(node_features, batch):
    partials = _sc_partial_sums(node_features, batch.astype(jnp.int32))
    return _combine(partials)
